# baseline (speedup 1.0000x reference)
import dataclasses

import jax
import jax.numpy as jnp
import numpy as np
from jax import lax
from jax.experimental import pallas as pl
from jax.experimental.pallas import tpu as pltpu
from jax.experimental.pallas import tpu_sc as plsc

B = 16384
EPS = 1e-5
NC, NS = 2, 16
NW = NC * NS
IPT = B // NW
DEPTH = 4
SLOT = 50

_SC_COMPILER_PARAMS = pltpu.CompilerParams(use_tc_tiling_on_sc=False)
if "needs_layout_passes" in pltpu.CompilerParams.__dataclass_fields__:
    _SC_COMPILER_PARAMS = dataclasses.replace(
        _SC_COMPILER_PARAMS, needs_layout_passes=False)


def _mesh():
    return plsc.VectorSubcoreMesh(core_axis_name="c", subcore_axis_name="s")


def _sc_bag(idx, tab, width):
    dtype = tab.dtype
    scratch_types = [
        pltpu.VMEM((IPT, width), jnp.int32),
        pltpu.VMEM((DEPTH * SLOT, 128), dtype),
        pltpu.VMEM((32, 128), jnp.float32),
        pltpu.SemaphoreType.DMA,
        pltpu.SemaphoreType.DMA,
        pltpu.SemaphoreType.DMA,
        pltpu.SemaphoreType.DMA,
    ]

    def body(idx_r, tab_r, out_o, idx_v, gbag, stage, s0, s1, s2, s3):
        wid = lax.axis_index("s") * NC + lax.axis_index("c")
        base = wid * IPT
        sems = [s0, s1, s2, s3]

        pltpu.sync_copy(idx_r.at[pl.ds(base, IPT)], idx_v)

        def slot_ref(slot):
            return gbag.at[pl.ds(slot * SLOT, width)]

        def issue(j, slot):
            pltpu.async_copy(tab_r.at[idx_v.at[j]], slot_ref(slot),
                             sems[slot])

        for slot in range(DEPTH):
            issue(slot, slot)

        @pl.loop(0, IPT // DEPTH)
        def _(jo):
            jb = jo * DEPTH
            for slot in range(DEPTH):
                j = jb + slot
                pltpu.make_async_copy(tab_r.at[idx_v.at[j]],
                                      slot_ref(slot), sems[slot]).wait()
                srow = slot * SLOT
                row = lax.rem(jo, 8) * DEPTH + slot
                if dtype == jnp.bfloat16:
                    for g in range(4):
                        cs = pl.ds(g * 32, 32)
                        parts = []
                        r = 0
                        while r + 4 <= width:
                            parts.append(
                                (gbag[srow + r, cs] + gbag[srow + r + 1, cs])
                                + (gbag[srow + r + 2, cs]
                                   + gbag[srow + r + 3, cs]))
                            r += 4
                        if r + 2 <= width:
                            parts.append(gbag[srow + r, cs]
                                         + gbag[srow + r + 1, cs])
                            r += 2
                        if r < width:
                            parts.append(gbag[srow + r, cs])
                        acc_e, acc_o = plsc.unpack(
                            parts[0], format=plsc.PackFormat.INTERLEAVED)
                        for p in parts[1:]:
                            e, o = plsc.unpack(
                                p, format=plsc.PackFormat.INTERLEAVED)
                            acc_e = acc_e + e
                            acc_o = acc_o + o
                        stage[row, pl.ds(g * 32, 16)] = acc_e
                        stage[row, pl.ds(g * 32 + 16, 16)] = acc_o
                else:
                    for g in range(8):
                        cs = pl.ds(g * 16, 16)
                        parts = [gbag[srow + r, cs] + gbag[srow + r + 1, cs]
                                 for r in range(0, width - 1, 2)]
                        if width % 2:
                            parts.append(gbag[srow + width - 1, cs])
                        acc = parts[0]
                        for p in parts[1:]:
                            acc = acc + p
                        stage[row, cs] = acc

                nxt = j + DEPTH

                @pl.when(nxt < IPT)
                def _():
                    issue(nxt, slot)

            @pl.when(lax.rem(jo, 8) == 7)
            def _():
                pltpu.sync_copy(
                    stage, out_o.at[pl.ds(base + jb - 28, 32)])

    run = pl.kernel(body,
                    out_type=jax.ShapeDtypeStruct((B, 128), jnp.float32),
                    mesh=_mesh(), scratch_types=scratch_types,
                    compiler_params=_SC_COMPILER_PARAMS)
    return run(idx, tab)


def _sc_smalls(bidx, cidx, coidx, c1i, c2i, c3i,
               btab, ctab, cotab, t1, t2, t3):
    f32 = jnp.float32
    out_type = [
        jax.ShapeDtypeStruct((B, 64), f32),
        jax.ShapeDtypeStruct((B, 32), f32),
        jax.ShapeDtypeStruct((B, 16), f32),
        jax.ShapeDtypeStruct((B, 16), f32),
        jax.ShapeDtypeStruct((B, 16), f32),
        jax.ShapeDtypeStruct((B, 16), f32),
    ]
    scratch_types = [
        pltpu.VMEM((IPT,), jnp.int32),
        pltpu.VMEM((2, 128, 64), f32),
        pltpu.VMEM((2, 128, 32), f32),
        pltpu.VMEM((2, 128, 16), f32),
        pltpu.SemaphoreType.DMA,
        pltpu.SemaphoreType.DMA,
    ]

    def body(bidx_r, cidx_r, coidx_r, c1i_r, c2i_r, c3i_r,
             btab_r, ctab_r, cotab_r, t1_r, t2_r, t3_r,
             brand_o, cat_o, cond_o, c1o, c2o, c3o,
             sidx_v, gs64, gs32, gs16, s0, s1):
        wid = lax.axis_index("s") * NC + lax.axis_index("c")
        base = wid * IPT
        sems = [s0, s1]

        def plain(idx1d_r, tab_r, out_o, gdst):
            pltpu.sync_copy(idx1d_r.at[pl.ds(base, IPT)], sidx_v)

            def issue(k):
                pltpu.async_copy(tab_r.at[sidx_v.at[pl.ds(k * 128, 128)]],
                                 gdst.at[k % 2], sems[k % 2])

            issue(0)
            for k in range(4):
                pltpu.make_async_copy(
                    tab_r.at[sidx_v.at[pl.ds(k * 128, 128)]],
                    gdst.at[k % 2], sems[k % 2]).wait()
                if k + 1 < 4:
                    issue(k + 1)
                pltpu.sync_copy(gdst.at[k % 2],
                                out_o.at[pl.ds(base + k * 128, 128)])

        plain(bidx_r, btab_r, brand_o, gs64)
        plain(cidx_r, ctab_r, cat_o, gs32)
        plain(coidx_r, cotab_r, cond_o, gs16)
        plain(c1i_r, t1_r, c1o, gs16)
        plain(c2i_r, t2_r, c2o, gs16)
        plain(c3i_r, t3_r, c3o, gs16)

    run = pl.kernel(body, out_type=out_type, mesh=_mesh(),
                    scratch_types=scratch_types,
                    compiler_params=_SC_COMPILER_PARAMS)
    return run(bidx, cidx, coidx, c1i, c2i, c3i,
               btab, ctab, cotab, t1, t2, t3)


def _tc_mlp(name_e, text_e, brand_e, cat_e, cond_e, ship, c1_e, c2_e, c3_e,
            w1, b1, g1, be1, m1, v1, g2, be2, m2, v2, w2, b2):
    BLK = 2048
    grid = (B // BLK,)

    def body(n_r, t_r, br_r, ca_r, co_r, sh_r, x1_r, x2_r, x3_r,
             w1_r, b1_r, g1_r, be1_r, m1_r, v1_r,
             g2_r, be2_r, m2_r, v2_r, w2_r, b2_r, out_r):
        w1f = w1_r[...]
        s1 = g1_r[...] * lax.rsqrt(v1_r[...] + EPS)
        w1s = w1f * s1.reshape(417, 1)
        b1f = b1_r[...] + (be1_r[...] - m1_r[...] * s1) @ w1f

        f32 = jnp.float32
        z = jnp.dot(n_r[...], w1s[0:128], preferred_element_type=f32)
        z += jnp.dot(t_r[...], w1s[128:256], preferred_element_type=f32)
        z += jnp.dot(br_r[...], w1s[256:320], preferred_element_type=f32)
        z += jnp.dot(ca_r[...], w1s[320:352], preferred_element_type=f32)
        z += jnp.dot(co_r[...], w1s[352:368], preferred_element_type=f32)
        z += sh_r[...] * w1s[368:369]
        z += jnp.dot(x1_r[...], w1s[369:385], preferred_element_type=f32)
        z += jnp.dot(x2_r[...], w1s[385:401], preferred_element_type=f32)
        z += jnp.dot(x3_r[...], w1s[401:417], preferred_element_type=f32)
        z += b1f
        h = jnp.where(z > 0, z, 0.01 * z)

        w2f = w2_r[...]
        s2 = g2_r[...] * lax.rsqrt(v2_r[...] + EPS)
        w2s = w2f * s2.reshape(150, 1)
        b2f = b2_r[...] + (be2_r[...] - m2_r[...] * s2) @ w2f
        out_r[...] = jnp.dot(h, w2s, preferred_element_type=f32) + b2f

    row_spec = lambda w: pl.BlockSpec((BLK, w), lambda i: (i, 0))
    full = lambda a: pl.BlockSpec(a.shape, lambda i: (0,) * a.ndim)

    return pl.pallas_call(
        body,
        grid=grid,
        in_specs=[
            row_spec(128), row_spec(128), row_spec(64), row_spec(32),
            row_spec(16), row_spec(1), row_spec(16), row_spec(16),
            row_spec(16),
            full(w1), full(b1), full(g1), full(be1), full(m1), full(v1),
            full(g2), full(be2), full(m2), full(v2), full(w2), full(b2),
        ],
        out_specs=pl.BlockSpec((BLK, 1), lambda i: (i, 0)),
        out_shape=jax.ShapeDtypeStruct((B, 1), jnp.float32),
    )(name_e, text_e, brand_e, cat_e, cond_e, ship, c1_e, c2_e, c3_e,
      w1, b1, g1, be1, m1, v1, g2, be2, m2, v2, w2, b2)


def kernel(item_name, text_description, brand_name, category,
           item_condition_id, shipping_flag, cat1, cat2, cat3,
           name_table, text_table, brand_table, category_table,
           condition_table, cat1_table, cat2_table, cat3_table,
           bn1_gamma, bn1_beta, bn1_mean, bn1_var, fc1_w, fc1_b,
           bn2_gamma, bn2_beta, bn2_mean, bn2_var, fc2_w, fc2_b):
    i32 = jnp.int32

    brand_e, cat_e, cond_e, c1_e, c2_e, c3_e = _sc_smalls(
        brand_name.astype(i32), category.astype(i32),
        item_condition_id.astype(i32), cat1.astype(i32),
        cat2.astype(i32), cat3.astype(i32),
        brand_table, category_table, condition_table,
        cat1_table, cat2_table, cat3_table)
    name_e = _sc_bag(item_name.astype(i32), name_table, 20)
    zero = brand_e[0, 0] * 0.0
    text_e = _sc_bag(text_description.astype(i32),
                     (text_table + zero).astype(jnp.bfloat16), 50)

    seg = np.array([g * 32 + 2 * k + h
                    for g in range(4) for h in range(2) for k in range(16)])
    perm = np.concatenate([np.arange(128), 128 + seg, np.arange(256, 417)])
    fc1_wp = fc1_w[perm]
    pv = lambda a: a[perm].reshape(1, -1)
    nsc = np.concatenate([np.full(128, 20.0), np.full(128, 50.0),
                          np.ones(161)]).astype(np.float32)
    bn1_gamma = bn1_gamma / nsc
    bn1_mean = bn1_mean * nsc

    r1 = lambda a: a.reshape(1, -1)
    return _tc_mlp(
        name_e, text_e, brand_e, cat_e, cond_e, shipping_flag,
        c1_e, c2_e, c3_e,
        fc1_wp, r1(fc1_b), pv(bn1_gamma), pv(bn1_beta), pv(bn1_mean),
        pv(bn1_var), r1(bn2_gamma), r1(bn2_beta), r1(bn2_mean),
        r1(bn2_var), fc2_w, r1(fc2_b))


# speedup vs baseline: 1.0530x; 1.0530x over previous
import dataclasses

import jax
import jax.numpy as jnp
import numpy as np
from jax import lax
from jax.experimental import pallas as pl
from jax.experimental.pallas import tpu as pltpu
from jax.experimental.pallas import tpu_sc as plsc

B = 16384
EPS = 1e-5
NC, NS = 2, 16
NW = NC * NS
IPT = B // NW
DEPTH = 4
SLOT = 50

_SC_COMPILER_PARAMS = pltpu.CompilerParams(use_tc_tiling_on_sc=False)
if "needs_layout_passes" in pltpu.CompilerParams.__dataclass_fields__:
    _SC_COMPILER_PARAMS = dataclasses.replace(
        _SC_COMPILER_PARAMS, needs_layout_passes=False)


def _mesh():
    return plsc.VectorSubcoreMesh(core_axis_name="c", subcore_axis_name="s")


def _sc_bag(idx, tab, width):
    dtype = tab.dtype
    scratch_types = [
        pltpu.VMEM((IPT, width), jnp.int32),
        pltpu.VMEM((DEPTH * SLOT, 128), dtype),
        pltpu.VMEM((32, 128), jnp.float32),
        pltpu.SemaphoreType.DMA,
        pltpu.SemaphoreType.DMA,
        pltpu.SemaphoreType.DMA,
        pltpu.SemaphoreType.DMA,
    ]

    def body(idx_r, tab_r, out_o, idx_v, gbag, stage, s0, s1, s2, s3):
        wid = lax.axis_index("s") * NC + lax.axis_index("c")
        base = wid * IPT
        sems = [s0, s1, s2, s3]

        pltpu.sync_copy(idx_r.at[pl.ds(base, IPT)], idx_v)

        def slot_ref(slot):
            return gbag.at[pl.ds(slot * SLOT, width)]

        def issue(j, slot):
            pltpu.async_copy(tab_r.at[idx_v.at[j]], slot_ref(slot),
                             sems[slot])

        for slot in range(DEPTH):
            issue(slot, slot)

        @pl.loop(0, IPT // DEPTH)
        def _(jo):
            jb = jo * DEPTH
            for slot in range(DEPTH):
                j = jb + slot
                pltpu.make_async_copy(tab_r.at[idx_v.at[j]],
                                      slot_ref(slot), sems[slot]).wait()
                srow = slot * SLOT
                row = lax.rem(jo, 8) * DEPTH + slot
                if dtype == jnp.bfloat16:
                    for g in range(4):
                        cs = pl.ds(g * 32, 32)
                        parts = []
                        r = 0
                        while r + 4 <= width:
                            parts.append(
                                (gbag[srow + r, cs] + gbag[srow + r + 1, cs])
                                + (gbag[srow + r + 2, cs]
                                   + gbag[srow + r + 3, cs]))
                            r += 4
                        if r + 2 <= width:
                            parts.append(gbag[srow + r, cs]
                                         + gbag[srow + r + 1, cs])
                            r += 2
                        if r < width:
                            parts.append(gbag[srow + r, cs])
                        acc_e, acc_o = plsc.unpack(
                            parts[0], format=plsc.PackFormat.INTERLEAVED)
                        for p in parts[1:]:
                            e, o = plsc.unpack(
                                p, format=plsc.PackFormat.INTERLEAVED)
                            acc_e = acc_e + e
                            acc_o = acc_o + o
                        stage[row, pl.ds(g * 32, 16)] = acc_e
                        stage[row, pl.ds(g * 32 + 16, 16)] = acc_o
                else:
                    for g in range(8):
                        cs = pl.ds(g * 16, 16)
                        parts = [gbag[srow + r, cs] + gbag[srow + r + 1, cs]
                                 for r in range(0, width - 1, 2)]
                        if width % 2:
                            parts.append(gbag[srow + width - 1, cs])
                        acc = parts[0]
                        for p in parts[1:]:
                            acc = acc + p
                        stage[row, cs] = acc

                nxt = j + DEPTH

                @pl.when(nxt < IPT)
                def _():
                    issue(nxt, slot)

            @pl.when(lax.rem(jo, 8) == 7)
            def _():
                pltpu.sync_copy(
                    stage, out_o.at[pl.ds(base + jb - 28, 32)])

    run = pl.kernel(body,
                    out_type=jax.ShapeDtypeStruct((B, 128), jnp.float32),
                    mesh=_mesh(), scratch_types=scratch_types,
                    compiler_params=_SC_COMPILER_PARAMS)
    return run(idx, tab)


def _sc_smalls(bidx, cidx, coidx, c1i, c2i, c3i,
               btab, ctab, cotab, t1, t2, t3):
    f32 = jnp.float32
    out_type = [
        jax.ShapeDtypeStruct((B, 64), f32),
        jax.ShapeDtypeStruct((B, 32), f32),
        jax.ShapeDtypeStruct((B, 16), f32),
        jax.ShapeDtypeStruct((B, 16), f32),
        jax.ShapeDtypeStruct((B, 16), f32),
        jax.ShapeDtypeStruct((B, 16), f32),
    ]
    scratch_types = [
        pltpu.VMEM((IPT,), jnp.int32),
        pltpu.VMEM((2, 128, 64), f32),
        pltpu.VMEM((2, 128, 32), f32),
        pltpu.VMEM((2, 128, 16), f32),
        pltpu.SemaphoreType.DMA,
        pltpu.SemaphoreType.DMA,
    ]

    def body(bidx_r, cidx_r, coidx_r, c1i_r, c2i_r, c3i_r,
             btab_r, ctab_r, cotab_r, t1_r, t2_r, t3_r,
             brand_o, cat_o, cond_o, c1o, c2o, c3o,
             sidx_v, gs64, gs32, gs16, s0, s1):
        wid = lax.axis_index("s") * NC + lax.axis_index("c")
        base = wid * IPT
        sems = [s0, s1]

        def plain(idx1d_r, tab_r, out_o, gdst):
            pltpu.sync_copy(idx1d_r.at[pl.ds(base, IPT)], sidx_v)

            def issue(k):
                pltpu.async_copy(tab_r.at[sidx_v.at[pl.ds(k * 128, 128)]],
                                 gdst.at[k % 2], sems[k % 2])

            issue(0)
            for k in range(4):
                pltpu.make_async_copy(
                    tab_r.at[sidx_v.at[pl.ds(k * 128, 128)]],
                    gdst.at[k % 2], sems[k % 2]).wait()
                if k + 1 < 4:
                    issue(k + 1)
                pltpu.sync_copy(gdst.at[k % 2],
                                out_o.at[pl.ds(base + k * 128, 128)])

        plain(bidx_r, btab_r, brand_o, gs64)
        plain(cidx_r, ctab_r, cat_o, gs32)
        plain(coidx_r, cotab_r, cond_o, gs16)
        plain(c1i_r, t1_r, c1o, gs16)
        plain(c2i_r, t2_r, c2o, gs16)
        plain(c3i_r, t3_r, c3o, gs16)

    run = pl.kernel(body, out_type=out_type, mesh=_mesh(),
                    scratch_types=scratch_types,
                    compiler_params=_SC_COMPILER_PARAMS)
    return run(bidx, cidx, coidx, c1i, c2i, c3i,
               btab, ctab, cotab, t1, t2, t3)


BLK = 2048
_row_spec = lambda w: pl.BlockSpec((BLK, w), lambda i: (i, 0))
_full = lambda a: pl.BlockSpec(a.shape, lambda i: (0,) * a.ndim)


def _tc_mlp_partial(name_e, brand_e, cat_e, cond_e, ship, c1_e, c2_e, c3_e,
                    w1, b1, g1, be1, m1, v1):
    def body(n_r, br_r, ca_r, co_r, sh_r, x1_r, x2_r, x3_r,
             w1_r, b1_r, g1_r, be1_r, m1_r, v1_r, zp_r):
        w1f = w1_r[...]
        s1 = g1_r[...] * lax.rsqrt(v1_r[...] + EPS)
        w1s = w1f * s1.reshape(417, 1)
        b1f = b1_r[...] + (be1_r[...] - m1_r[...] * s1) @ w1f

        f32 = jnp.float32
        z = jnp.dot(n_r[...], w1s[0:128], preferred_element_type=f32)
        z += jnp.dot(br_r[...], w1s[256:320], preferred_element_type=f32)
        z += jnp.dot(ca_r[...], w1s[320:352], preferred_element_type=f32)
        z += jnp.dot(co_r[...], w1s[352:368], preferred_element_type=f32)
        z += sh_r[...] * w1s[368:369]
        z += jnp.dot(x1_r[...], w1s[369:385], preferred_element_type=f32)
        z += jnp.dot(x2_r[...], w1s[385:401], preferred_element_type=f32)
        z += jnp.dot(x3_r[...], w1s[401:417], preferred_element_type=f32)
        zp_r[...] = z + b1f

    return pl.pallas_call(
        body,
        grid=(B // BLK,),
        in_specs=[
            _row_spec(128), _row_spec(64), _row_spec(32), _row_spec(16),
            _row_spec(1), _row_spec(16), _row_spec(16), _row_spec(16),
            _full(w1), _full(b1), _full(g1), _full(be1), _full(m1),
            _full(v1),
        ],
        out_specs=pl.BlockSpec((BLK, 150), lambda i: (i, 0)),
        out_shape=jax.ShapeDtypeStruct((B, 150), jnp.float32),
    )(name_e, brand_e, cat_e, cond_e, ship, c1_e, c2_e, c3_e,
      w1, b1, g1, be1, m1, v1)


def _tc_mlp_final(zp, text_e, w1, g1, v1, g2, be2, m2, v2, w2, b2):
    def body(zp_r, t_r, w1_r, g1_r, v1_r,
             g2_r, be2_r, m2_r, v2_r, w2_r, b2_r, out_r):
        f32 = jnp.float32
        s1 = g1_r[...] * lax.rsqrt(v1_r[...] + EPS)
        w1s_t = w1_r[...][128:256] * s1[0, 128:256].reshape(128, 1)
        z = zp_r[...] + jnp.dot(t_r[...], w1s_t, preferred_element_type=f32)
        h = jnp.where(z > 0, z, 0.01 * z)

        w2f = w2_r[...]
        s2 = g2_r[...] * lax.rsqrt(v2_r[...] + EPS)
        w2s = w2f * s2.reshape(150, 1)
        b2f = b2_r[...] + (be2_r[...] - m2_r[...] * s2) @ w2f
        out_r[...] = jnp.dot(h, w2s, preferred_element_type=f32) + b2f

    return pl.pallas_call(
        body,
        grid=(B // BLK,),
        in_specs=[
            pl.BlockSpec((BLK, 150), lambda i: (i, 0)), _row_spec(128),
            _full(w1), _full(g1), _full(v1),
            _full(g2), _full(be2), _full(m2), _full(v2), _full(w2),
            _full(b2),
        ],
        out_specs=pl.BlockSpec((BLK, 1), lambda i: (i, 0)),
        out_shape=jax.ShapeDtypeStruct((B, 1), jnp.float32),
    )(zp, text_e, w1, g1, v1, g2, be2, m2, v2, w2, b2)


def kernel(item_name, text_description, brand_name, category,
           item_condition_id, shipping_flag, cat1, cat2, cat3,
           name_table, text_table, brand_table, category_table,
           condition_table, cat1_table, cat2_table, cat3_table,
           bn1_gamma, bn1_beta, bn1_mean, bn1_var, fc1_w, fc1_b,
           bn2_gamma, bn2_beta, bn2_mean, bn2_var, fc2_w, fc2_b):
    i32 = jnp.int32

    brand_e, cat_e, cond_e, c1_e, c2_e, c3_e = _sc_smalls(
        brand_name.astype(i32), category.astype(i32),
        item_condition_id.astype(i32), cat1.astype(i32),
        cat2.astype(i32), cat3.astype(i32),
        brand_table, category_table, condition_table,
        cat1_table, cat2_table, cat3_table)
    name_e = _sc_bag(item_name.astype(i32), name_table, 20)
    text_e = _sc_bag(text_description.astype(i32),
                     text_table.astype(jnp.bfloat16), 50)

    seg = np.array([g * 32 + 2 * k + h
                    for g in range(4) for h in range(2) for k in range(16)])
    perm = np.concatenate([np.arange(128), 128 + seg, np.arange(256, 417)])
    fc1_wp = fc1_w[perm]
    pv = lambda a: a[perm].reshape(1, -1)
    nsc = np.concatenate([np.full(128, 20.0), np.full(128, 50.0),
                          np.ones(161)]).astype(np.float32)
    bn1_gamma = bn1_gamma / nsc
    bn1_mean = bn1_mean * nsc

    r1 = lambda a: a.reshape(1, -1)
    zp = _tc_mlp_partial(
        name_e, brand_e, cat_e, cond_e, shipping_flag, c1_e, c2_e, c3_e,
        fc1_wp, r1(fc1_b), pv(bn1_gamma), pv(bn1_beta), pv(bn1_mean),
        pv(bn1_var))
    return _tc_mlp_final(
        zp, text_e, fc1_wp, pv(bn1_gamma), pv(bn1_var),
        r1(bn2_gamma), r1(bn2_beta), r1(bn2_mean), r1(bn2_var),
        fc2_w, r1(fc2_b))


# speedup vs baseline: 1.0557x; 1.0025x over previous
import dataclasses

import jax
import jax.numpy as jnp
import numpy as np
from jax import lax
from jax.experimental import pallas as pl
from jax.experimental.pallas import tpu as pltpu
from jax.experimental.pallas import tpu_sc as plsc

B = 16384
EPS = 1e-5
NC, NS = 2, 16
NW = NC * NS
IPT = B // NW
DEPTH = 4
SLOT = 50

_SC_COMPILER_PARAMS = pltpu.CompilerParams(use_tc_tiling_on_sc=False)
if "needs_layout_passes" in pltpu.CompilerParams.__dataclass_fields__:
    _SC_COMPILER_PARAMS = dataclasses.replace(
        _SC_COMPILER_PARAMS, needs_layout_passes=False)


def _mesh():
    return plsc.VectorSubcoreMesh(core_axis_name="c", subcore_axis_name="s")


def _sc_bag(idx, tab, width):
    dtype = tab.dtype
    scratch_types = [
        pltpu.VMEM((IPT, width), jnp.int32),
        pltpu.VMEM((DEPTH * SLOT, 128), dtype),
        pltpu.VMEM((32, 128), jnp.float32),
        pltpu.SemaphoreType.DMA,
        pltpu.SemaphoreType.DMA,
        pltpu.SemaphoreType.DMA,
        pltpu.SemaphoreType.DMA,
    ]

    def body(idx_r, tab_r, out_o, idx_v, gbag, stage, s0, s1, s2, s3):
        wid = lax.axis_index("s") * NC + lax.axis_index("c")
        base = wid * IPT
        sems = [s0, s1, s2, s3]

        pltpu.sync_copy(idx_r.at[pl.ds(base, IPT)], idx_v)

        def slot_ref(slot):
            return gbag.at[pl.ds(slot * SLOT, width)]

        def issue(j, slot):
            pltpu.async_copy(tab_r.at[idx_v.at[j]], slot_ref(slot),
                             sems[slot])

        for slot in range(DEPTH):
            issue(slot, slot)

        @pl.loop(0, IPT // DEPTH)
        def _(jo):
            jb = jo * DEPTH
            for slot in range(DEPTH):
                j = jb + slot
                pltpu.make_async_copy(tab_r.at[idx_v.at[j]],
                                      slot_ref(slot), sems[slot]).wait()
                srow = slot * SLOT
                row = lax.rem(jo, 8) * DEPTH + slot
                if dtype == jnp.bfloat16:
                    for g in range(4):
                        cs = pl.ds(g * 32, 32)
                        parts = []
                        r = 0
                        while r + 4 <= width:
                            parts.append(
                                (gbag[srow + r, cs] + gbag[srow + r + 1, cs])
                                + (gbag[srow + r + 2, cs]
                                   + gbag[srow + r + 3, cs]))
                            r += 4
                        if r + 2 <= width:
                            parts.append(gbag[srow + r, cs]
                                         + gbag[srow + r + 1, cs])
                            r += 2
                        if r < width:
                            parts.append(gbag[srow + r, cs])
                        acc_e, acc_o = plsc.unpack(
                            parts[0], format=plsc.PackFormat.INTERLEAVED)
                        for p in parts[1:]:
                            e, o = plsc.unpack(
                                p, format=plsc.PackFormat.INTERLEAVED)
                            acc_e = acc_e + e
                            acc_o = acc_o + o
                        stage[row, pl.ds(g * 32, 16)] = acc_e
                        stage[row, pl.ds(g * 32 + 16, 16)] = acc_o
                else:
                    for g in range(8):
                        cs = pl.ds(g * 16, 16)
                        parts = [gbag[srow + r, cs] + gbag[srow + r + 1, cs]
                                 for r in range(0, width - 1, 2)]
                        if width % 2:
                            parts.append(gbag[srow + width - 1, cs])
                        acc = parts[0]
                        for p in parts[1:]:
                            acc = acc + p
                        stage[row, cs] = acc

                nxt = j + DEPTH

                @pl.when(nxt < IPT)
                def _():
                    issue(nxt, slot)

            @pl.when(lax.rem(jo, 8) == 7)
            def _():
                pltpu.sync_copy(
                    stage, out_o.at[pl.ds(base + jb - 28, 32)])

    run = pl.kernel(body,
                    out_type=jax.ShapeDtypeStruct((B, 128), jnp.float32),
                    mesh=_mesh(), scratch_types=scratch_types,
                    compiler_params=_SC_COMPILER_PARAMS)
    return run(idx, tab)


def _sc_smalls(bidx, cidx, coidx, c1i, c2i, c3i,
               btab, ctab, cotab, t1, t2, t3):
    f32 = jnp.float32
    out_type = [
        jax.ShapeDtypeStruct((B, 64), f32),
        jax.ShapeDtypeStruct((B, 32), f32),
        jax.ShapeDtypeStruct((B, 16), f32),
        jax.ShapeDtypeStruct((B, 16), f32),
        jax.ShapeDtypeStruct((B, 16), f32),
        jax.ShapeDtypeStruct((B, 16), f32),
    ]
    scratch_types = [
        pltpu.VMEM((IPT,), jnp.int32),
        pltpu.VMEM((2, 128, 128), f32),
        pltpu.VMEM((2, 128, 32), f32),
        pltpu.VMEM((2, 128, 16), f32),
        pltpu.SemaphoreType.DMA,
        pltpu.SemaphoreType.DMA,
    ]

    def body(bidx_r, cidx_r, coidx_r, c1i_r, c2i_r, c3i_r,
             btab_r, ctab_r, cotab_r, t1_r, t2_r, t3_r,
             brand_o, cat_o, cond_o, c1o, c2o, c3o,
             sidx_v, gs64, gs32, gs16, s0, s1):
        wid = lax.axis_index("s") * NC + lax.axis_index("c")
        base = wid * IPT
        sems = [s0, s1]

        def plain(idx1d_r, tab_r, out_o, gdst, out_w=None):
            pltpu.sync_copy(idx1d_r.at[pl.ds(base, IPT)], sidx_v)

            def issue(k):
                pltpu.async_copy(tab_r.at[sidx_v.at[pl.ds(k * 128, 128)]],
                                 gdst.at[k % 2], sems[k % 2])

            issue(0)
            for k in range(4):
                pltpu.make_async_copy(
                    tab_r.at[sidx_v.at[pl.ds(k * 128, 128)]],
                    gdst.at[k % 2], sems[k % 2]).wait()
                if k + 1 < 4:
                    issue(k + 1)
                src = gdst.at[k % 2]
                if out_w is not None:
                    src = src.at[:, pl.ds(0, out_w)]
                pltpu.sync_copy(src, out_o.at[pl.ds(base + k * 128, 128)])

        plain(bidx_r, btab_r, brand_o, gs64, out_w=64)
        plain(cidx_r, ctab_r, cat_o, gs32)
        plain(coidx_r, cotab_r, cond_o, gs16)
        plain(c1i_r, t1_r, c1o, gs16)
        plain(c2i_r, t2_r, c2o, gs16)
        plain(c3i_r, t3_r, c3o, gs16)

    run = pl.kernel(body, out_type=out_type, mesh=_mesh(),
                    scratch_types=scratch_types,
                    compiler_params=_SC_COMPILER_PARAMS)
    return run(bidx, cidx, coidx, c1i, c2i, c3i,
               btab, ctab, cotab, t1, t2, t3)


BLK = 2048
_row_spec = lambda w: pl.BlockSpec((BLK, w), lambda i: (i, 0))
_full = lambda a: pl.BlockSpec(a.shape, lambda i: (0,) * a.ndim)


def _tc_mlp_partial(name_e, brand_e, cat_e, cond_e, ship, c1_e, c2_e, c3_e,
                    w1, b1, g1, be1, m1, v1):
    def body(n_r, br_r, ca_r, co_r, sh_r, x1_r, x2_r, x3_r,
             w1_r, b1_r, g1_r, be1_r, m1_r, v1_r, zp_r):
        w1f = w1_r[...]
        s1 = g1_r[...] * lax.rsqrt(v1_r[...] + EPS)
        w1s = w1f * s1.reshape(417, 1)
        b1f = b1_r[...] + (be1_r[...] - m1_r[...] * s1) @ w1f

        f32 = jnp.float32
        z = jnp.dot(n_r[...], w1s[0:128], preferred_element_type=f32)
        z += jnp.dot(br_r[...], w1s[256:320], preferred_element_type=f32)
        z += jnp.dot(ca_r[...], w1s[320:352], preferred_element_type=f32)
        z += jnp.dot(co_r[...], w1s[352:368], preferred_element_type=f32)
        z += sh_r[...] * w1s[368:369]
        z += jnp.dot(x1_r[...], w1s[369:385], preferred_element_type=f32)
        z += jnp.dot(x2_r[...], w1s[385:401], preferred_element_type=f32)
        z += jnp.dot(x3_r[...], w1s[401:417], preferred_element_type=f32)
        zp_r[...] = z + b1f

    return pl.pallas_call(
        body,
        grid=(B // BLK,),
        in_specs=[
            _row_spec(128), _row_spec(64), _row_spec(32), _row_spec(16),
            _row_spec(1), _row_spec(16), _row_spec(16), _row_spec(16),
            _full(w1), _full(b1), _full(g1), _full(be1), _full(m1),
            _full(v1),
        ],
        out_specs=pl.BlockSpec((BLK, 150), lambda i: (i, 0)),
        out_shape=jax.ShapeDtypeStruct((B, 150), jnp.float32),
    )(name_e, brand_e, cat_e, cond_e, ship, c1_e, c2_e, c3_e,
      w1, b1, g1, be1, m1, v1)


def _tc_mlp_final(zp, text_e, w1, g1, v1, g2, be2, m2, v2, w2, b2):
    def body(zp_r, t_r, w1_r, g1_r, v1_r,
             g2_r, be2_r, m2_r, v2_r, w2_r, b2_r, out_r):
        f32 = jnp.float32
        s1 = g1_r[...] * lax.rsqrt(v1_r[...] + EPS)
        w1s_t = w1_r[...][128:256] * s1[0, 128:256].reshape(128, 1)
        z = zp_r[...] + jnp.dot(t_r[...], w1s_t, preferred_element_type=f32)
        h = jnp.where(z > 0, z, 0.01 * z)

        w2f = w2_r[...]
        s2 = g2_r[...] * lax.rsqrt(v2_r[...] + EPS)
        w2s = w2f * s2.reshape(150, 1)
        b2f = b2_r[...] + (be2_r[...] - m2_r[...] * s2) @ w2f
        out_r[...] = jnp.dot(h, w2s, preferred_element_type=f32) + b2f

    return pl.pallas_call(
        body,
        grid=(B // BLK,),
        in_specs=[
            pl.BlockSpec((BLK, 150), lambda i: (i, 0)), _row_spec(128),
            _full(w1), _full(g1), _full(v1),
            _full(g2), _full(be2), _full(m2), _full(v2), _full(w2),
            _full(b2),
        ],
        out_specs=pl.BlockSpec((BLK, 1), lambda i: (i, 0)),
        out_shape=jax.ShapeDtypeStruct((B, 1), jnp.float32),
    )(zp, text_e, w1, g1, v1, g2, be2, m2, v2, w2, b2)


def kernel(item_name, text_description, brand_name, category,
           item_condition_id, shipping_flag, cat1, cat2, cat3,
           name_table, text_table, brand_table, category_table,
           condition_table, cat1_table, cat2_table, cat3_table,
           bn1_gamma, bn1_beta, bn1_mean, bn1_var, fc1_w, fc1_b,
           bn2_gamma, bn2_beta, bn2_mean, bn2_var, fc2_w, fc2_b):
    i32 = jnp.int32

    brand_e, cat_e, cond_e, c1_e, c2_e, c3_e = _sc_smalls(
        brand_name.astype(i32), category.astype(i32),
        item_condition_id.astype(i32), cat1.astype(i32),
        cat2.astype(i32), cat3.astype(i32),
        jnp.pad(brand_table, ((0, 0), (0, 64))),
        category_table, condition_table,
        cat1_table, cat2_table, cat3_table)
    name_e = _sc_bag(item_name.astype(i32), name_table, 20)
    text_e = _sc_bag(text_description.astype(i32),
                     text_table.astype(jnp.bfloat16), 50)

    seg = np.array([g * 32 + 2 * k + h
                    for g in range(4) for h in range(2) for k in range(16)])
    perm = np.concatenate([np.arange(128), 128 + seg, np.arange(256, 417)])
    fc1_wp = fc1_w[perm]
    pv = lambda a: a[perm].reshape(1, -1)
    nsc = np.concatenate([np.full(128, 20.0), np.full(128, 50.0),
                          np.ones(161)]).astype(np.float32)
    bn1_gamma = bn1_gamma / nsc
    bn1_mean = bn1_mean * nsc

    r1 = lambda a: a.reshape(1, -1)
    zp = _tc_mlp_partial(
        name_e, brand_e, cat_e, cond_e, shipping_flag, c1_e, c2_e, c3_e,
        fc1_wp, r1(fc1_b), pv(bn1_gamma), pv(bn1_beta), pv(bn1_mean),
        pv(bn1_var))
    return _tc_mlp_final(
        zp, text_e, fc1_wp, pv(bn1_gamma), pv(bn1_var),
        r1(bn2_gamma), r1(bn2_beta), r1(bn2_mean), r1(bn2_var),
        fc2_w, r1(fc2_b))
